# baseline (speedup 1.0000x reference)
import functools

import jax
import jax.numpy as jnp
from jax import lax
from jax.experimental import pallas as pl
from jax.experimental.pallas import tpu as pltpu
from jax.experimental.pallas import tpu_sc as plsc

VOCAB = 1000000
EMBED = 64
BATCH = 16384
NEG = 5

NC = 2
NS = 16
NW = NC * NS
BPW = BATCH // NW
CHUNK = 128
NCH = BPW // CHUNK
GROUPS = CHUNK // 16
NT = 1 + NEG
TROW = 17
EPAD = 128
TBLK = 2048
HBLK = TBLK // 2
TGRID = (VOCAB + TBLK - 1) // TBLK
TTAIL = VOCAB - (TGRID - 1) * TBLK
PROWS = (TGRID - 1) * HBLK + TTAIL


def _tc_relayout(wt):
  grid = TGRID

  def body(x_ref, o_hbm, buf0, buf1, sem0, sem1):
    i = pl.program_id(0)

    def run(buf, sem):
      @pl.when(i >= 2)
      def _():
        pltpu.make_async_copy(
            buf, o_hbm.at[pl.ds((i - 2) * HBLK, HBLK)], sem).wait()

      xt = x_ref[...].T
      buf[...] = jnp.concatenate([xt[:HBLK], xt[HBLK:]], axis=1)

      @pl.when(i < grid - 1)
      def _():
        pltpu.make_async_copy(
            buf, o_hbm.at[pl.ds(i * HBLK, HBLK)], sem).start()

      @pl.when(i == grid - 1)
      def _():
        pltpu.make_async_copy(
            buf.at[pl.ds(0, TTAIL)],
            o_hbm.at[pl.ds(i * HBLK, TTAIL)], sem).start()

    @pl.when(i % 2 == 0)
    def _():
      run(buf0, sem0)

    @pl.when(i % 2 == 1)
    def _():
      run(buf1, sem1)

    @pl.when(i == grid - 1)
    def _():
      pltpu.make_async_copy(
          buf1, o_hbm.at[pl.ds((grid - 2) * HBLK, HBLK)], sem1).wait()
      pltpu.make_async_copy(
          buf0.at[pl.ds(0, TTAIL)],
          o_hbm.at[pl.ds((grid - 1) * HBLK, TTAIL)], sem0).wait()

  assert (grid - 1) % 2 == 0 and TTAIL % 8 == 0 and TTAIL <= HBLK
  return pl.pallas_call(
      body,
      grid=(grid,),
      in_specs=[pl.BlockSpec((EMBED, TBLK), lambda i: (0, i))],
      out_specs=pl.BlockSpec(memory_space=pl.ANY),
      out_shape=jax.ShapeDtypeStruct((PROWS, EPAD), jnp.float32),
      scratch_shapes=[
          pltpu.VMEM((HBLK, EPAD), jnp.float32),
          pltpu.VMEM((HBLK, EPAD), jnp.float32),
          pltpu.SemaphoreType.DMA,
          pltpu.SemaphoreType.DMA,
      ],
  )(wt)


def _sc_scores(center, context, neg_flat, w_center, w_context):
  mesh = plsc.VectorSubcoreMesh(core_axis_name="c", subcore_axis_name="s",
                                num_cores=NC, num_subcores=NS)

  @functools.partial(
      pl.kernel,
      out_type=jax.ShapeDtypeStruct((BATCH * NT,), jnp.float32),
      mesh=mesh,
      compiler_params=pltpu.CompilerParams(needs_layout_passes=False,
                                           use_tc_tiling_on_sc=True),
      scratch_types=[
          pltpu.VMEM((CHUNK + 16,), jnp.int32),
          pltpu.VMEM((CHUNK + 16,), jnp.int32),
          pltpu.VMEM((CHUNK * NEG + 16,), jnp.int32),
          pltpu.VMEM((CHUNK,), jnp.int32),
          pltpu.VMEM((CHUNK,), jnp.int32),
          pltpu.VMEM((CHUNK * NEG,), jnp.int32),
          pltpu.VMEM((CHUNK, EPAD), jnp.float32),
          pltpu.VMEM((CHUNK, EPAD), jnp.float32),
          pltpu.VMEM((CHUNK * NEG, EPAD), jnp.float32),
          pltpu.VMEM((NT * 16 * TROW,), jnp.float32),
          pltpu.VMEM((NT * CHUNK,), jnp.float32),
          pltpu.SemaphoreType.DMA,
      ],
  )
  def k(center_hbm, context_hbm, neg_hbm, wcp_hbm, wxp_hbm, out_hbm,
        cidx, xidx, nidx, cri, xri, nri, crows, xrows, nrows, tscr, sbuf, sem):
    wid = lax.axis_index("s") * NC + lax.axis_index("c")
    lanes = lax.iota(jnp.int32, 16)

    def to_rows(src, dst, n16):
      def body(i, c):
        w = src[pl.ds(i * 16, 16)]
        dst[pl.ds(i * 16, 16)] = ((w >> 11) << 10) + (w & (HBLK - 1))
        return c
      lax.fori_loop(0, n16, body, 0)

    def chunk_body(ch, carry):
      base = wid * BPW + ch * CHUNK
      pltpu.sync_copy(center_hbm.at[pl.ds(base, CHUNK)],
                      cidx.at[pl.ds(0, CHUNK)])
      pltpu.sync_copy(context_hbm.at[pl.ds(base, CHUNK)],
                      xidx.at[pl.ds(0, CHUNK)])
      pltpu.sync_copy(neg_hbm.at[pl.ds(base * NEG, CHUNK * NEG)],
                      nidx.at[pl.ds(0, CHUNK * NEG)])
      to_rows(cidx, cri, CHUNK // 16)
      to_rows(xidx, xri, CHUNK // 16)
      to_rows(nidx, nri, CHUNK * NEG // 16)
      copies = [
          pltpu.async_copy(wcp_hbm.at[cri], crows, sem),
          pltpu.async_copy(wxp_hbm.at[xri], xrows, sem),
      ]
      for j in range(NEG):
        copies.append(pltpu.async_copy(
            wxp_hbm.at[nri.at[pl.ds(j * CHUNK, CHUNK)]],
            nrows.at[pl.ds(j * CHUNK, CHUNK)], sem))
      for cp in copies:
        cp.wait()

      def group_body(g, carry2):
        for e in range(16):
          b = g * 16 + e
          oc = ((cidx[pl.ds(b, 16)][0] >> 10) & 1) * EMBED
          ox = ((xidx[pl.ds(b, 16)][0] >> 10) & 1) * EMBED
          cvs = [crows[b, pl.ds(oc + j * 16, 16)] for j in range(4)]
          xvs = [xrows[b, pl.ds(ox + j * 16, 16)] for j in range(4)]
          p = cvs[0] * xvs[0]
          for j in range(1, 4):
            p = p + cvs[j] * xvs[j]
          tscr[pl.ds(0 * 16 * TROW + e * TROW, 16)] = p
          for t in range(NEG):
            on = ((nidx[pl.ds(b * NEG + t, 16)][0] >> 10) & 1) * EMBED
            nvs = [nrows[b * NEG + t, pl.ds(on + j * 16, 16)] for j in range(4)]
            q = cvs[0] * nvs[0]
            for j in range(1, 4):
              q = q + cvs[j] * nvs[j]
            tscr[pl.ds((t + 1) * 16 * TROW + e * TROW, 16)] = q
        for t in range(NT):
          s = plsc.load_gather(tscr, [lanes * TROW + t * 16 * TROW])
          for d in range(1, 16):
            s = s + plsc.load_gather(tscr, [lanes * TROW + (t * 16 * TROW + d)])
          if t > 0:
            s = -s
          sbuf[pl.ds(t * CHUNK + g * 16, 16)] = s
        return carry2

      lax.fori_loop(0, GROUPS, group_body, 0)
      pltpu.sync_copy(
          sbuf, out_hbm.at[pl.ds(wid * BPW * NT + ch * CHUNK * NT, CHUNK * NT)])
      return carry

    lax.fori_loop(0, NCH, chunk_body, 0)

  return k(center, context, neg_flat, w_center, w_context)


def _tc_loss(scores):
  x2 = scores.reshape(NT * BATCH // 128, 128)

  def body(x_ref, o_ref):
    x = x_ref[...]
    ls = jnp.minimum(x, 0.0) - jnp.log1p(jnp.exp(-jnp.abs(x)))
    o_ref[0, 0] = -jnp.sum(ls) / BATCH

  out = pl.pallas_call(
      body,
      out_shape=jax.ShapeDtypeStruct((1, 1), jnp.float32),
      out_specs=pl.BlockSpec(memory_space=pltpu.SMEM),
  )(x2)
  return out[0, 0]


@jax.jit
def kernel(center_words, context_words, negative_words, W_center, W_context):
  center = jnp.asarray(center_words, jnp.int32)
  context = jnp.asarray(context_words, jnp.int32)
  neg_flat = jnp.asarray(negative_words, jnp.int32).reshape(-1)
  wc = _tc_relayout(W_center.T)
  wx = _tc_relayout(W_context.T)
  scores = _sc_scores(center, context, neg_flat, wc, wx)
  return _tc_loss(scores)


# speedup vs baseline: 1.1380x; 1.1380x over previous
import functools

import jax
import jax.numpy as jnp
from jax import lax
from jax.experimental import pallas as pl
from jax.experimental.pallas import tpu as pltpu
from jax.experimental.pallas import tpu_sc as plsc

VOCAB = 1000000
EMBED = 64
BATCH = 16384
NEG = 5

NC = 2
NS = 16
NW = NC * NS
BPW = BATCH // NW
CHUNK = 128
NCH = BPW // CHUNK
GROUPS = CHUNK // 16
NT = 1 + NEG
TROW = 17
EPAD = 128
TBLK = 2048
HBLK = TBLK // 2
TGRID = (VOCAB + TBLK - 1) // TBLK
TTAIL = VOCAB - (TGRID - 1) * TBLK
PROWS = (TGRID - 1) * HBLK + TTAIL


def _tc_relayout(wt):
  grid = TGRID

  def body(x_ref, o_hbm, buf0, buf1, sem0, sem1):
    i = pl.program_id(0)

    def run(buf, sem):
      @pl.when(i >= 2)
      def _():
        pltpu.make_async_copy(
            buf, o_hbm.at[pl.ds((i - 2) * HBLK, HBLK)], sem).wait()

      x2 = jnp.concatenate(
          [x_ref[:, : HBLK], x_ref[:, HBLK:]], axis=0)
      buf[...] = x2.T

      @pl.when(i < grid - 1)
      def _():
        pltpu.make_async_copy(
            buf, o_hbm.at[pl.ds(i * HBLK, HBLK)], sem).start()

      @pl.when(i == grid - 1)
      def _():
        pltpu.make_async_copy(
            buf.at[pl.ds(0, TTAIL)],
            o_hbm.at[pl.ds(i * HBLK, TTAIL)], sem).start()

    @pl.when(i % 2 == 0)
    def _():
      run(buf0, sem0)

    @pl.when(i % 2 == 1)
    def _():
      run(buf1, sem1)

    @pl.when(i == grid - 1)
    def _():
      pltpu.make_async_copy(
          buf1, o_hbm.at[pl.ds((grid - 2) * HBLK, HBLK)], sem1).wait()
      pltpu.make_async_copy(
          buf0.at[pl.ds(0, TTAIL)],
          o_hbm.at[pl.ds((grid - 1) * HBLK, TTAIL)], sem0).wait()

  assert (grid - 1) % 2 == 0 and TTAIL % 8 == 0 and TTAIL <= HBLK
  return pl.pallas_call(
      body,
      grid=(grid,),
      in_specs=[pl.BlockSpec((EMBED, TBLK), lambda i: (0, i))],
      out_specs=pl.BlockSpec(memory_space=pl.ANY),
      out_shape=jax.ShapeDtypeStruct((PROWS, EPAD), jnp.float32),
      scratch_shapes=[
          pltpu.VMEM((HBLK, EPAD), jnp.float32),
          pltpu.VMEM((HBLK, EPAD), jnp.float32),
          pltpu.SemaphoreType.DMA,
          pltpu.SemaphoreType.DMA,
      ],
  )(wt)


def _sc_scores(center, context, neg_flat, w_center, w_context):
  mesh = plsc.VectorSubcoreMesh(core_axis_name="c", subcore_axis_name="s",
                                num_cores=NC, num_subcores=NS)

  @functools.partial(
      pl.kernel,
      out_type=jax.ShapeDtypeStruct((BATCH * NT,), jnp.float32),
      mesh=mesh,
      compiler_params=pltpu.CompilerParams(needs_layout_passes=False,
                                           use_tc_tiling_on_sc=True),
      scratch_types=[
          pltpu.VMEM((CHUNK + 16,), jnp.int32),
          pltpu.VMEM((CHUNK + 16,), jnp.int32),
          pltpu.VMEM((CHUNK * NEG + 16,), jnp.int32),
          pltpu.VMEM((CHUNK,), jnp.int32),
          pltpu.VMEM((CHUNK,), jnp.int32),
          pltpu.VMEM((CHUNK * NEG,), jnp.int32),
          pltpu.VMEM((CHUNK, EPAD), jnp.float32),
          pltpu.VMEM((CHUNK, EPAD), jnp.float32),
          pltpu.VMEM((CHUNK * NEG, EPAD), jnp.float32),
          pltpu.VMEM((NT * 16 * TROW,), jnp.float32),
          pltpu.VMEM((NT * CHUNK,), jnp.float32),
          pltpu.SemaphoreType.DMA,
      ],
  )
  def k(center_hbm, context_hbm, neg_hbm, wcp_hbm, wxp_hbm, out_hbm,
        cidx, xidx, nidx, cri, xri, nri, crows, xrows, nrows, tscr, sbuf, sem):
    wid = lax.axis_index("s") * NC + lax.axis_index("c")
    lanes = lax.iota(jnp.int32, 16)

    def to_rows(src, dst, n16):
      def body(i, c):
        w = src[pl.ds(i * 16, 16)]
        dst[pl.ds(i * 16, 16)] = ((w >> 11) << 10) + (w & (HBLK - 1))
        return c
      lax.fori_loop(0, n16, body, 0)

    def chunk_body(ch, carry):
      base = wid * BPW + ch * CHUNK
      pltpu.sync_copy(center_hbm.at[pl.ds(base, CHUNK)],
                      cidx.at[pl.ds(0, CHUNK)])
      pltpu.sync_copy(context_hbm.at[pl.ds(base, CHUNK)],
                      xidx.at[pl.ds(0, CHUNK)])
      pltpu.sync_copy(neg_hbm.at[pl.ds(base * NEG, CHUNK * NEG)],
                      nidx.at[pl.ds(0, CHUNK * NEG)])
      to_rows(cidx, cri, CHUNK // 16)
      to_rows(xidx, xri, CHUNK // 16)
      to_rows(nidx, nri, CHUNK * NEG // 16)
      copies = [
          pltpu.async_copy(wcp_hbm.at[cri], crows, sem),
          pltpu.async_copy(wxp_hbm.at[xri], xrows, sem),
      ]
      for j in range(NEG):
        copies.append(pltpu.async_copy(
            wxp_hbm.at[nri.at[pl.ds(j * CHUNK, CHUNK)]],
            nrows.at[pl.ds(j * CHUNK, CHUNK)], sem))
      for cp in copies:
        cp.wait()

      def group_body(g, carry2):
        for e in range(16):
          b = g * 16 + e
          oc = ((cidx[pl.ds(b, 16)][0] >> 10) & 1) * EMBED
          ox = ((xidx[pl.ds(b, 16)][0] >> 10) & 1) * EMBED
          cvs = [crows[b, pl.ds(oc + j * 16, 16)] for j in range(4)]
          xvs = [xrows[b, pl.ds(ox + j * 16, 16)] for j in range(4)]
          p = cvs[0] * xvs[0]
          for j in range(1, 4):
            p = p + cvs[j] * xvs[j]
          tscr[pl.ds(0 * 16 * TROW + e * TROW, 16)] = p
          for t in range(NEG):
            on = ((nidx[pl.ds(b * NEG + t, 16)][0] >> 10) & 1) * EMBED
            nvs = [nrows[b * NEG + t, pl.ds(on + j * 16, 16)] for j in range(4)]
            q = cvs[0] * nvs[0]
            for j in range(1, 4):
              q = q + cvs[j] * nvs[j]
            tscr[pl.ds((t + 1) * 16 * TROW + e * TROW, 16)] = q
        for t in range(NT):
          s = plsc.load_gather(tscr, [lanes * TROW + t * 16 * TROW])
          for d in range(1, 16):
            s = s + plsc.load_gather(tscr, [lanes * TROW + (t * 16 * TROW + d)])
          if t > 0:
            s = -s
          sbuf[pl.ds(t * CHUNK + g * 16, 16)] = s
        return carry2

      lax.fori_loop(0, GROUPS, group_body, 0)
      pltpu.sync_copy(
          sbuf, out_hbm.at[pl.ds(wid * BPW * NT + ch * CHUNK * NT, CHUNK * NT)])
      return carry

    lax.fori_loop(0, NCH, chunk_body, 0)

  return k(center, context, neg_flat, w_center, w_context)


def _tc_loss(scores):
  x2 = scores.reshape(NT * BATCH // 128, 128)

  def body(x_ref, o_ref):
    x = x_ref[...]
    ls = jnp.minimum(x, 0.0) - jnp.log1p(jnp.exp(-jnp.abs(x)))
    o_ref[0, 0] = -jnp.sum(ls) / BATCH

  out = pl.pallas_call(
      body,
      out_shape=jax.ShapeDtypeStruct((1, 1), jnp.float32),
      out_specs=pl.BlockSpec(memory_space=pltpu.SMEM),
  )(x2)
  return out[0, 0]


@jax.jit
def kernel(center_words, context_words, negative_words, W_center, W_context):
  center = jnp.asarray(center_words, jnp.int32)
  context = jnp.asarray(context_words, jnp.int32)
  neg_flat = jnp.asarray(negative_words, jnp.int32).reshape(-1)
  wc = _tc_relayout(W_center.T)
  wx = _tc_relayout(W_context.T)
  scores = _sc_scores(center, context, neg_flat, wc, wx)
  return _tc_loss(scores)


# speedup vs baseline: 1.9272x; 1.6935x over previous
import functools

import jax
import jax.numpy as jnp
from jax import lax
from jax.experimental import pallas as pl
from jax.experimental.pallas import tpu as pltpu
from jax.experimental.pallas import tpu_sc as plsc

VOCAB = 1000000
EMBED = 64
BATCH = 16384
NEG = 5

NC = 2
NS = 16
NW = NC * NS
BPW = BATCH // NW
CHUNK = 128
NCH = BPW // CHUNK
GROUPS = CHUNK // 16
NT = 1 + NEG
TROW = 17
EPAD = 128
TBLK = 8192
HBLK = TBLK // 2
TSH = TBLK.bit_length() - 1
HSH = TSH - 1
TGRID = (VOCAB + TBLK - 1) // TBLK
TTAIL = VOCAB - (TGRID - 1) * TBLK
PROWS = (TGRID - 1) * HBLK + TTAIL


def _tc_relayout(wt):
  grid = TGRID

  def body(x_ref, o_hbm, buf0, buf1, sem0, sem1):
    i = pl.program_id(0)

    def run(buf, sem):
      @pl.when(i >= 2)
      def _():
        pltpu.make_async_copy(
            buf, o_hbm.at[pl.ds((i - 2) * HBLK, HBLK)], sem).wait()

      x2 = jnp.concatenate(
          [x_ref[:, : HBLK], x_ref[:, HBLK:]], axis=0)
      buf[...] = x2.T

      @pl.when(i < grid - 1)
      def _():
        pltpu.make_async_copy(
            buf, o_hbm.at[pl.ds(i * HBLK, HBLK)], sem).start()

      @pl.when(i == grid - 1)
      def _():
        pltpu.make_async_copy(
            buf.at[pl.ds(0, TTAIL)],
            o_hbm.at[pl.ds(i * HBLK, TTAIL)], sem).start()

    @pl.when(i % 2 == 0)
    def _():
      run(buf0, sem0)

    @pl.when(i % 2 == 1)
    def _():
      run(buf1, sem1)

    @pl.when(i == grid - 1)
    def _():
      pltpu.make_async_copy(
          buf1, o_hbm.at[pl.ds((grid - 2) * HBLK, HBLK)], sem1).wait()
      pltpu.make_async_copy(
          buf0.at[pl.ds(0, TTAIL)],
          o_hbm.at[pl.ds((grid - 1) * HBLK, TTAIL)], sem0).wait()

  assert (grid - 1) % 2 == 0 and TTAIL % 8 == 0 and TTAIL <= HBLK
  return pl.pallas_call(
      body,
      grid=(grid,),
      in_specs=[pl.BlockSpec((EMBED, TBLK), lambda i: (0, i))],
      out_specs=pl.BlockSpec(memory_space=pl.ANY),
      out_shape=jax.ShapeDtypeStruct((PROWS, EPAD), jnp.float32),
      scratch_shapes=[
          pltpu.VMEM((HBLK, EPAD), jnp.float32),
          pltpu.VMEM((HBLK, EPAD), jnp.float32),
          pltpu.SemaphoreType.DMA,
          pltpu.SemaphoreType.DMA,
      ],
  )(wt)


def _sc_scores(center, context, neg_flat, w_center, w_context):
  mesh = plsc.VectorSubcoreMesh(core_axis_name="c", subcore_axis_name="s",
                                num_cores=NC, num_subcores=NS)

  @functools.partial(
      pl.kernel,
      out_type=jax.ShapeDtypeStruct((BATCH * NT,), jnp.float32),
      mesh=mesh,
      compiler_params=pltpu.CompilerParams(needs_layout_passes=False,
                                           use_tc_tiling_on_sc=True),
      scratch_types=[
          pltpu.VMEM((CHUNK + 16,), jnp.int32),
          pltpu.VMEM((CHUNK + 16,), jnp.int32),
          pltpu.VMEM((CHUNK * NEG + 16,), jnp.int32),
          pltpu.VMEM((CHUNK,), jnp.int32),
          pltpu.VMEM((CHUNK,), jnp.int32),
          pltpu.VMEM((CHUNK * NEG,), jnp.int32),
          pltpu.VMEM((CHUNK, EPAD), jnp.float32),
          pltpu.VMEM((CHUNK, EPAD), jnp.float32),
          pltpu.VMEM((CHUNK * NEG, EPAD), jnp.float32),
          pltpu.VMEM((NT * 16 * TROW,), jnp.float32),
          pltpu.VMEM((NT * CHUNK,), jnp.float32),
          pltpu.SemaphoreType.DMA,
      ],
  )
  def k(center_hbm, context_hbm, neg_hbm, wcp_hbm, wxp_hbm, out_hbm,
        cidx, xidx, nidx, cri, xri, nri, crows, xrows, nrows, tscr, sbuf, sem):
    wid = lax.axis_index("s") * NC + lax.axis_index("c")
    lanes = lax.iota(jnp.int32, 16)

    def to_rows(src, dst, n16):
      def body(i, c):
        w = src[pl.ds(i * 16, 16)]
        dst[pl.ds(i * 16, 16)] = ((w >> TSH) << HSH) + (w & (HBLK - 1))
        return c
      lax.fori_loop(0, n16, body, 0)

    def chunk_body(ch, carry):
      base = wid * BPW + ch * CHUNK
      pltpu.sync_copy(center_hbm.at[pl.ds(base, CHUNK)],
                      cidx.at[pl.ds(0, CHUNK)])
      pltpu.sync_copy(context_hbm.at[pl.ds(base, CHUNK)],
                      xidx.at[pl.ds(0, CHUNK)])
      pltpu.sync_copy(neg_hbm.at[pl.ds(base * NEG, CHUNK * NEG)],
                      nidx.at[pl.ds(0, CHUNK * NEG)])
      to_rows(cidx, cri, CHUNK // 16)
      to_rows(xidx, xri, CHUNK // 16)
      to_rows(nidx, nri, CHUNK * NEG // 16)
      copies = [
          pltpu.async_copy(wcp_hbm.at[cri], crows, sem),
          pltpu.async_copy(wxp_hbm.at[xri], xrows, sem),
      ]
      for j in range(NEG):
        copies.append(pltpu.async_copy(
            wxp_hbm.at[nri.at[pl.ds(j * CHUNK, CHUNK)]],
            nrows.at[pl.ds(j * CHUNK, CHUNK)], sem))
      for cp in copies:
        cp.wait()

      def group_body(g, carry2):
        for e in range(16):
          b = g * 16 + e
          oc = ((cidx[pl.ds(b, 16)][0] >> HSH) & 1) * EMBED
          ox = ((xidx[pl.ds(b, 16)][0] >> HSH) & 1) * EMBED
          cvs = [crows[b, pl.ds(oc + j * 16, 16)] for j in range(4)]
          xvs = [xrows[b, pl.ds(ox + j * 16, 16)] for j in range(4)]
          p = cvs[0] * xvs[0]
          for j in range(1, 4):
            p = p + cvs[j] * xvs[j]
          tscr[pl.ds(0 * 16 * TROW + e * TROW, 16)] = p
          for t in range(NEG):
            on = ((nidx[pl.ds(b * NEG + t, 16)][0] >> HSH) & 1) * EMBED
            nvs = [nrows[b * NEG + t, pl.ds(on + j * 16, 16)] for j in range(4)]
            q = cvs[0] * nvs[0]
            for j in range(1, 4):
              q = q + cvs[j] * nvs[j]
            tscr[pl.ds((t + 1) * 16 * TROW + e * TROW, 16)] = q
        for t in range(NT):
          s = plsc.load_gather(tscr, [lanes * TROW + t * 16 * TROW])
          for d in range(1, 16):
            s = s + plsc.load_gather(tscr, [lanes * TROW + (t * 16 * TROW + d)])
          if t > 0:
            s = -s
          sbuf[pl.ds(t * CHUNK + g * 16, 16)] = s
        return carry2

      lax.fori_loop(0, GROUPS, group_body, 0)
      pltpu.sync_copy(
          sbuf, out_hbm.at[pl.ds(wid * BPW * NT + ch * CHUNK * NT, CHUNK * NT)])
      return carry

    lax.fori_loop(0, NCH, chunk_body, 0)

  return k(center, context, neg_flat, w_center, w_context)


def _tc_loss(scores):
  x2 = scores.reshape(NT * BATCH // 128, 128)

  def body(x_ref, o_ref):
    x = x_ref[...]
    ls = jnp.minimum(x, 0.0) - jnp.log1p(jnp.exp(-jnp.abs(x)))
    o_ref[0, 0] = -jnp.sum(ls) / BATCH

  out = pl.pallas_call(
      body,
      out_shape=jax.ShapeDtypeStruct((1, 1), jnp.float32),
      out_specs=pl.BlockSpec(memory_space=pltpu.SMEM),
  )(x2)
  return out[0, 0]


@jax.jit
def kernel(center_words, context_words, negative_words, W_center, W_context):
  center = jnp.asarray(center_words, jnp.int32)
  context = jnp.asarray(context_words, jnp.int32)
  neg_flat = jnp.asarray(negative_words, jnp.int32).reshape(-1)
  wc = _tc_relayout(W_center.T)
  wx = _tc_relayout(W_context.T)
  scores = _sc_scores(center, context, neg_flat, wc, wx)
  return _tc_loss(scores)


# speedup vs baseline: 2.1483x; 1.1147x over previous
import functools

import jax
import jax.numpy as jnp
from jax import lax
from jax.experimental import pallas as pl
from jax.experimental.pallas import tpu as pltpu
from jax.experimental.pallas import tpu_sc as plsc

VOCAB = 1000000
EMBED = 64
BATCH = 16384
NEG = 5

NC = 2
NS = 16
NW = NC * NS
BPW = BATCH // NW
CHUNK = 128
NCH = BPW // CHUNK
GROUPS = CHUNK // 16
NT = 1 + NEG
TROW = 17
EPAD = 128
TBLK = 16384
HBLK = TBLK // 2
TSH = TBLK.bit_length() - 1
HSH = TSH - 1
TGRID = (VOCAB + TBLK - 1) // TBLK
TTAIL = VOCAB - (TGRID - 1) * TBLK
PROWS = (TGRID - 1) * HBLK + TTAIL


def _tc_relayout(wt):
  grid = TGRID

  def body(x_ref, o_hbm, buf0, buf1, sem0, sem1):
    i = pl.program_id(0)

    def run(buf, sem):
      @pl.when(i >= 2)
      def _():
        pltpu.make_async_copy(
            buf, o_hbm.at[pl.ds((i - 2) * HBLK, HBLK)], sem).wait()

      x2 = jnp.concatenate(
          [x_ref[:, : HBLK], x_ref[:, HBLK:]], axis=0)
      buf[...] = x2.T

      @pl.when(i < grid - 1)
      def _():
        pltpu.make_async_copy(
            buf, o_hbm.at[pl.ds(i * HBLK, HBLK)], sem).start()

      @pl.when(i == grid - 1)
      def _():
        pltpu.make_async_copy(
            buf.at[pl.ds(0, TTAIL)],
            o_hbm.at[pl.ds(i * HBLK, TTAIL)], sem).start()

    @pl.when(i % 2 == 0)
    def _():
      run(buf0, sem0)

    @pl.when(i % 2 == 1)
    def _():
      run(buf1, sem1)

    last, prev = ((buf0, sem0), (buf1, sem1))
    if (grid - 1) % 2 == 1:
      last, prev = prev, last

    @pl.when(i == grid - 1)
    def _():
      pltpu.make_async_copy(
          prev[0], o_hbm.at[pl.ds((grid - 2) * HBLK, HBLK)], prev[1]).wait()
      pltpu.make_async_copy(
          last[0].at[pl.ds(0, TTAIL)],
          o_hbm.at[pl.ds((grid - 1) * HBLK, TTAIL)], last[1]).wait()

  assert TTAIL % 8 == 0 and TTAIL <= HBLK
  return pl.pallas_call(
      body,
      grid=(grid,),
      in_specs=[pl.BlockSpec((EMBED, TBLK), lambda i: (0, i))],
      out_specs=pl.BlockSpec(memory_space=pl.ANY),
      out_shape=jax.ShapeDtypeStruct((PROWS, EPAD), jnp.float32),
      scratch_shapes=[
          pltpu.VMEM((HBLK, EPAD), jnp.float32),
          pltpu.VMEM((HBLK, EPAD), jnp.float32),
          pltpu.SemaphoreType.DMA,
          pltpu.SemaphoreType.DMA,
      ],
  )(wt)


def _sc_scores(center, context, neg_flat, w_center, w_context):
  mesh = plsc.VectorSubcoreMesh(core_axis_name="c", subcore_axis_name="s",
                                num_cores=NC, num_subcores=NS)

  @functools.partial(
      pl.kernel,
      out_type=jax.ShapeDtypeStruct((BATCH * NT,), jnp.float32),
      mesh=mesh,
      compiler_params=pltpu.CompilerParams(needs_layout_passes=False,
                                           use_tc_tiling_on_sc=True),
      scratch_types=[
          pltpu.VMEM((CHUNK + 16,), jnp.int32),
          pltpu.VMEM((CHUNK + 16,), jnp.int32),
          pltpu.VMEM((CHUNK * NEG + 16,), jnp.int32),
          pltpu.VMEM((CHUNK,), jnp.int32),
          pltpu.VMEM((CHUNK,), jnp.int32),
          pltpu.VMEM((CHUNK * NEG,), jnp.int32),
          pltpu.VMEM((CHUNK, EPAD), jnp.float32),
          pltpu.VMEM((CHUNK, EPAD), jnp.float32),
          pltpu.VMEM((CHUNK * NEG, EPAD), jnp.float32),
          pltpu.VMEM((NT * 16 * TROW,), jnp.float32),
          pltpu.VMEM((NT * CHUNK,), jnp.float32),
          pltpu.SemaphoreType.DMA,
      ],
  )
  def k(center_hbm, context_hbm, neg_hbm, wcp_hbm, wxp_hbm, out_hbm,
        cidx, xidx, nidx, cri, xri, nri, crows, xrows, nrows, tscr, sbuf, sem):
    wid = lax.axis_index("s") * NC + lax.axis_index("c")
    lanes = lax.iota(jnp.int32, 16)

    def to_rows(src, dst, n16):
      def body(i, c):
        w = src[pl.ds(i * 16, 16)]
        dst[pl.ds(i * 16, 16)] = ((w >> TSH) << HSH) + (w & (HBLK - 1))
        return c
      lax.fori_loop(0, n16, body, 0)

    def chunk_body(ch, carry):
      base = wid * BPW + ch * CHUNK
      pltpu.sync_copy(center_hbm.at[pl.ds(base, CHUNK)],
                      cidx.at[pl.ds(0, CHUNK)])
      pltpu.sync_copy(context_hbm.at[pl.ds(base, CHUNK)],
                      xidx.at[pl.ds(0, CHUNK)])
      pltpu.sync_copy(neg_hbm.at[pl.ds(base * NEG, CHUNK * NEG)],
                      nidx.at[pl.ds(0, CHUNK * NEG)])
      to_rows(cidx, cri, CHUNK // 16)
      to_rows(xidx, xri, CHUNK // 16)
      to_rows(nidx, nri, CHUNK * NEG // 16)
      copies = [
          pltpu.async_copy(wcp_hbm.at[cri], crows, sem),
          pltpu.async_copy(wxp_hbm.at[xri], xrows, sem),
      ]
      for j in range(NEG):
        copies.append(pltpu.async_copy(
            wxp_hbm.at[nri.at[pl.ds(j * CHUNK, CHUNK)]],
            nrows.at[pl.ds(j * CHUNK, CHUNK)], sem))
      for cp in copies:
        cp.wait()

      def group_body(g, carry2):
        for e in range(16):
          b = g * 16 + e
          oc = ((cidx[pl.ds(b, 16)][0] >> HSH) & 1) * EMBED
          ox = ((xidx[pl.ds(b, 16)][0] >> HSH) & 1) * EMBED
          cvs = [crows[b, pl.ds(oc + j * 16, 16)] for j in range(4)]
          xvs = [xrows[b, pl.ds(ox + j * 16, 16)] for j in range(4)]
          p = cvs[0] * xvs[0]
          for j in range(1, 4):
            p = p + cvs[j] * xvs[j]
          tscr[pl.ds(0 * 16 * TROW + e * TROW, 16)] = p
          for t in range(NEG):
            on = ((nidx[pl.ds(b * NEG + t, 16)][0] >> HSH) & 1) * EMBED
            nvs = [nrows[b * NEG + t, pl.ds(on + j * 16, 16)] for j in range(4)]
            q = cvs[0] * nvs[0]
            for j in range(1, 4):
              q = q + cvs[j] * nvs[j]
            tscr[pl.ds((t + 1) * 16 * TROW + e * TROW, 16)] = q
        for t in range(NT):
          s = plsc.load_gather(tscr, [lanes * TROW + t * 16 * TROW])
          for d in range(1, 16):
            s = s + plsc.load_gather(tscr, [lanes * TROW + (t * 16 * TROW + d)])
          if t > 0:
            s = -s
          sbuf[pl.ds(t * CHUNK + g * 16, 16)] = s
        return carry2

      lax.fori_loop(0, GROUPS, group_body, 0)
      pltpu.sync_copy(
          sbuf, out_hbm.at[pl.ds(wid * BPW * NT + ch * CHUNK * NT, CHUNK * NT)])
      return carry

    lax.fori_loop(0, NCH, chunk_body, 0)

  return k(center, context, neg_flat, w_center, w_context)


def _tc_loss(scores):
  x2 = scores.reshape(NT * BATCH // 128, 128)

  def body(x_ref, o_ref):
    x = x_ref[...]
    ls = jnp.minimum(x, 0.0) - jnp.log1p(jnp.exp(-jnp.abs(x)))
    o_ref[0, 0] = -jnp.sum(ls) / BATCH

  out = pl.pallas_call(
      body,
      out_shape=jax.ShapeDtypeStruct((1, 1), jnp.float32),
      out_specs=pl.BlockSpec(memory_space=pltpu.SMEM),
  )(x2)
  return out[0, 0]


@jax.jit
def kernel(center_words, context_words, negative_words, W_center, W_context):
  center = jnp.asarray(center_words, jnp.int32)
  context = jnp.asarray(context_words, jnp.int32)
  neg_flat = jnp.asarray(negative_words, jnp.int32).reshape(-1)
  wc = _tc_relayout(W_center.T)
  wx = _tc_relayout(W_context.T)
  scores = _sc_scores(center, context, neg_flat, wc, wx)
  return _tc_loss(scores)


# speedup vs baseline: 2.1787x; 1.0142x over previous
import functools

import jax
import jax.numpy as jnp
from jax import lax
from jax.experimental import pallas as pl
from jax.experimental.pallas import tpu as pltpu
from jax.experimental.pallas import tpu_sc as plsc

VOCAB = 1000000
EMBED = 64
BATCH = 16384
NEG = 5

NC = 2
NS = 16
NW = NC * NS
BPW = BATCH // NW
CHUNK = 128
NCH = BPW // CHUNK
GROUPS = CHUNK // 16
NT = 1 + NEG
TROW = 17
EPAD = 128
TBLK = 32768
HBLK = TBLK // 2
TSH = TBLK.bit_length() - 1
HSH = TSH - 1
TGRID = (VOCAB + TBLK - 1) // TBLK
TTAIL = VOCAB - (TGRID - 1) * TBLK
TAILROWS = min(TTAIL, HBLK)
PROWS = (TGRID - 1) * HBLK + TAILROWS


def _tc_relayout(wt):
  grid = TGRID

  def body(x_ref, o_hbm, buf0, buf1, sem0, sem1):
    i = pl.program_id(0)

    def run(buf, sem):
      @pl.when(i >= 2)
      def _():
        pltpu.make_async_copy(
            buf, o_hbm.at[pl.ds((i - 2) * HBLK, HBLK)], sem).wait()

      x2 = jnp.concatenate(
          [x_ref[:, : HBLK], x_ref[:, HBLK:]], axis=0)
      buf[...] = x2.T

      @pl.when(i < grid - 1)
      def _():
        pltpu.make_async_copy(
            buf, o_hbm.at[pl.ds(i * HBLK, HBLK)], sem).start()

      @pl.when(i == grid - 1)
      def _():
        pltpu.make_async_copy(
            buf.at[pl.ds(0, TAILROWS)],
            o_hbm.at[pl.ds(i * HBLK, TAILROWS)], sem).start()

    @pl.when(i % 2 == 0)
    def _():
      run(buf0, sem0)

    @pl.when(i % 2 == 1)
    def _():
      run(buf1, sem1)

    last, prev = ((buf0, sem0), (buf1, sem1))
    if (grid - 1) % 2 == 1:
      last, prev = prev, last

    @pl.when(i == grid - 1)
    def _():
      pltpu.make_async_copy(
          prev[0], o_hbm.at[pl.ds((grid - 2) * HBLK, HBLK)], prev[1]).wait()
      pltpu.make_async_copy(
          last[0].at[pl.ds(0, TAILROWS)],
          o_hbm.at[pl.ds((grid - 1) * HBLK, TAILROWS)], last[1]).wait()

  assert TAILROWS % 8 == 0
  return pl.pallas_call(
      body,
      grid=(grid,),
      in_specs=[pl.BlockSpec((EMBED, TBLK), lambda i: (0, i))],
      out_specs=pl.BlockSpec(memory_space=pl.ANY),
      out_shape=jax.ShapeDtypeStruct((PROWS, EPAD), jnp.float32),
      scratch_shapes=[
          pltpu.VMEM((HBLK, EPAD), jnp.float32),
          pltpu.VMEM((HBLK, EPAD), jnp.float32),
          pltpu.SemaphoreType.DMA,
          pltpu.SemaphoreType.DMA,
      ],
  )(wt)


def _sc_scores(center, context, neg_flat, w_center, w_context):
  mesh = plsc.VectorSubcoreMesh(core_axis_name="c", subcore_axis_name="s",
                                num_cores=NC, num_subcores=NS)

  @functools.partial(
      pl.kernel,
      out_type=jax.ShapeDtypeStruct((BATCH * NT,), jnp.float32),
      mesh=mesh,
      compiler_params=pltpu.CompilerParams(needs_layout_passes=False,
                                           use_tc_tiling_on_sc=True),
      scratch_types=[
          pltpu.VMEM((CHUNK + 16,), jnp.int32),
          pltpu.VMEM((CHUNK + 16,), jnp.int32),
          pltpu.VMEM((CHUNK * NEG + 16,), jnp.int32),
          pltpu.VMEM((CHUNK,), jnp.int32),
          pltpu.VMEM((CHUNK,), jnp.int32),
          pltpu.VMEM((CHUNK * NEG,), jnp.int32),
          pltpu.VMEM((CHUNK, EPAD), jnp.float32),
          pltpu.VMEM((CHUNK, EPAD), jnp.float32),
          pltpu.VMEM((CHUNK * NEG, EPAD), jnp.float32),
          pltpu.VMEM((NT * 16 * TROW,), jnp.float32),
          pltpu.VMEM((NT * CHUNK,), jnp.float32),
          pltpu.SemaphoreType.DMA,
      ],
  )
  def k(center_hbm, context_hbm, neg_hbm, wcp_hbm, wxp_hbm, out_hbm,
        cidx, xidx, nidx, cri, xri, nri, crows, xrows, nrows, tscr, sbuf, sem):
    wid = lax.axis_index("s") * NC + lax.axis_index("c")
    lanes = lax.iota(jnp.int32, 16)

    def to_rows(src, dst, n16):
      def body(i, c):
        w = src[pl.ds(i * 16, 16)]
        dst[pl.ds(i * 16, 16)] = ((w >> TSH) << HSH) + (w & (HBLK - 1))
        return c
      lax.fori_loop(0, n16, body, 0)

    def chunk_body(ch, carry):
      base = wid * BPW + ch * CHUNK
      pltpu.sync_copy(center_hbm.at[pl.ds(base, CHUNK)],
                      cidx.at[pl.ds(0, CHUNK)])
      pltpu.sync_copy(context_hbm.at[pl.ds(base, CHUNK)],
                      xidx.at[pl.ds(0, CHUNK)])
      pltpu.sync_copy(neg_hbm.at[pl.ds(base * NEG, CHUNK * NEG)],
                      nidx.at[pl.ds(0, CHUNK * NEG)])
      to_rows(cidx, cri, CHUNK // 16)
      to_rows(xidx, xri, CHUNK // 16)
      to_rows(nidx, nri, CHUNK * NEG // 16)
      copies = [
          pltpu.async_copy(wcp_hbm.at[cri], crows, sem),
          pltpu.async_copy(wxp_hbm.at[xri], xrows, sem),
      ]
      for j in range(NEG):
        copies.append(pltpu.async_copy(
            wxp_hbm.at[nri.at[pl.ds(j * CHUNK, CHUNK)]],
            nrows.at[pl.ds(j * CHUNK, CHUNK)], sem))
      for cp in copies:
        cp.wait()

      def group_body(g, carry2):
        for e in range(16):
          b = g * 16 + e
          oc = ((cidx[pl.ds(b, 16)][0] >> HSH) & 1) * EMBED
          ox = ((xidx[pl.ds(b, 16)][0] >> HSH) & 1) * EMBED
          cvs = [crows[b, pl.ds(oc + j * 16, 16)] for j in range(4)]
          xvs = [xrows[b, pl.ds(ox + j * 16, 16)] for j in range(4)]
          p = cvs[0] * xvs[0]
          for j in range(1, 4):
            p = p + cvs[j] * xvs[j]
          tscr[pl.ds(0 * 16 * TROW + e * TROW, 16)] = p
          for t in range(NEG):
            on = ((nidx[pl.ds(b * NEG + t, 16)][0] >> HSH) & 1) * EMBED
            nvs = [nrows[b * NEG + t, pl.ds(on + j * 16, 16)] for j in range(4)]
            q = cvs[0] * nvs[0]
            for j in range(1, 4):
              q = q + cvs[j] * nvs[j]
            tscr[pl.ds((t + 1) * 16 * TROW + e * TROW, 16)] = q
        for t in range(NT):
          s = plsc.load_gather(tscr, [lanes * TROW + t * 16 * TROW])
          for d in range(1, 16):
            s = s + plsc.load_gather(tscr, [lanes * TROW + (t * 16 * TROW + d)])
          if t > 0:
            s = -s
          sbuf[pl.ds(t * CHUNK + g * 16, 16)] = s
        return carry2

      lax.fori_loop(0, GROUPS, group_body, 0)
      pltpu.sync_copy(
          sbuf, out_hbm.at[pl.ds(wid * BPW * NT + ch * CHUNK * NT, CHUNK * NT)])
      return carry

    lax.fori_loop(0, NCH, chunk_body, 0)

  return k(center, context, neg_flat, w_center, w_context)


def _tc_loss(scores):
  x2 = scores.reshape(NT * BATCH // 128, 128)

  def body(x_ref, o_ref):
    x = x_ref[...]
    ls = jnp.minimum(x, 0.0) - jnp.log1p(jnp.exp(-jnp.abs(x)))
    o_ref[0, 0] = -jnp.sum(ls) / BATCH

  out = pl.pallas_call(
      body,
      out_shape=jax.ShapeDtypeStruct((1, 1), jnp.float32),
      out_specs=pl.BlockSpec(memory_space=pltpu.SMEM),
  )(x2)
  return out[0, 0]


@jax.jit
def kernel(center_words, context_words, negative_words, W_center, W_context):
  center = jnp.asarray(center_words, jnp.int32)
  context = jnp.asarray(context_words, jnp.int32)
  neg_flat = jnp.asarray(negative_words, jnp.int32).reshape(-1)
  wc = _tc_relayout(W_center.T)
  wx = _tc_relayout(W_context.T)
  scores = _sc_scores(center, context, neg_flat, wc, wx)
  return _tc_loss(scores)
